# baseline (speedup 1.0000x reference)
import functools

import jax
import jax.numpy as jnp
from jax import lax
from jax.experimental import pallas as pl
from jax.experimental.pallas import tpu as pltpu
from jax.experimental.pallas import tpu_sc as plsc

_N = 10000
_E = 160000
_D = 128
_ED = 16

_NC = 2
_NS = 16
_NW = _NC * _NS
_ACC = 10240
_CHUNK = 128
_ROWS = _E // _CHUNK
_ROWS_PW = 40
_FIRE = 8
_ROWS_PAD = _NW * _ROWS_PW
_DEAD = _N + 16
_ZLEN = _ACC // _NS


def _deg_body(dst_hbm, out_hbm, idx_v, ones_v, zeros_v, acc_shared, sem):
    c = lax.axis_index("c")
    s = lax.axis_index("s")
    wid = c * _NS + s
    for i in range(_CHUNK // 16):
        ones_v[pl.ds(i * 16, 16)] = jnp.ones((16,), jnp.float32)
    for i in range(_ZLEN // 16):
        zeros_v[pl.ds(i * 16, 16)] = jnp.zeros((16,), jnp.float32)
    pltpu.sync_copy(dst_hbm.at[pl.ds(wid * _ROWS_PW, _ROWS_PW)], idx_v)
    pltpu.sync_copy(zeros_v, acc_shared.at[pl.ds(s * _ZLEN, _ZLEN)])
    plsc.subcore_barrier()

    def _block(b, carry):
        copies = [
            pltpu.async_copy(
                ones_v, acc_shared.at[idx_v.at[b * _FIRE + j]], sem, add=True)
            for j in range(_FIRE)
        ]
        for cp in copies:
            cp.wait()
        return carry

    lax.fori_loop(0, _ROWS_PW // _FIRE, _block, 0)
    plsc.subcore_barrier()
    pltpu.sync_copy(acc_shared.at[pl.ds(s * _ZLEN, _ZLEN)],
                    out_hbm.at[pl.ds(c * _ACC + s * _ZLEN, _ZLEN)])


@functools.cache
def _deg_counts_fn():
    return pl.kernel(
        _deg_body,
        out_type=jax.ShapeDtypeStruct((_NC * _ACC,), jnp.float32),
        mesh=plsc.VectorSubcoreMesh(core_axis_name="c", subcore_axis_name="s"),
        scratch_types=[
            pltpu.VMEM((_ROWS_PW, _CHUNK), jnp.int32),
            pltpu.VMEM((_CHUNK,), jnp.float32),
            pltpu.VMEM((_ZLEN,), jnp.float32),
            pltpu.VMEM_SHARED((_ACC,), jnp.float32),
            pltpu.SemaphoreType.DMA,
        ],
    )


_NODE_TILE = 2000
_NODE_STEPS = _N // _NODE_TILE
_EDGE_TILE_T = 32000
_EDGE_STEPS = _E // _EDGE_TILE_T
_CONTRACT_T = (((1,), (1,)), ((), ()))


def _fused_body(efT_ref, we_ref, be_ref, x_ref, deg_ref, watt_ref, wrel_ref,
                wout_ref, bout_ref, wfc_ref, bfc_ref, oe_ref, on_ref,
                wcomb_ref):
    i = pl.program_id(0)

    @pl.when(i < _EDGE_STEPS)
    def _edge():
        xe = efT_ref[...]
        oe_ref[...] = jnp.maximum(
            lax.dot_general(we_ref[...], xe, (((1,), (0,)), ((), ())),
                            preferred_element_type=jnp.float32)
            + be_ref[...], 0.0) + xe

    @pl.when(i == _EDGE_STEPS)
    def _fold():
        kk = watt_ref.shape[0]
        mm = wrel_ref.shape[0]
        wk = watt_ref[0]
        for k in range(1, kk):
            wk = wk + watt_ref[k]
        wm = wrel_ref[0]
        for m in range(1, mm):
            wm = wm + wrel_ref[m]
        c1 = wout_ref[:, :_D]
        c2 = wout_ref[:, _D:]
        wcomb_ref[...] = (
            lax.dot_general(wk, c1, _CONTRACT_T,
                            preferred_element_type=jnp.float32) / kk
            + lax.dot_general(wm, c2, _CONTRACT_T,
                              preferred_element_type=jnp.float32) / mm)

    @pl.when(i >= _EDGE_STEPS)
    def _node():
        h = x_ref[...]
        t = jnp.maximum(
            jnp.dot(h, wcomb_ref[...], preferred_element_type=jnp.float32)
            + bout_ref[...], 0.0)
        d = deg_ref[0] + deg_ref[1]
        h2 = jnp.where(d > 0.0, t, h)
        on_ref[...] = jnp.maximum(
            lax.dot_general(h2, wfc_ref[...], _CONTRACT_T,
                            preferred_element_type=jnp.float32)
            + bfc_ref[...], 0.0) + h


def _fused_call(efT, we, be, x, degs, watt, wrel, wout, bout, wfc, bfc):
    ecl = _EDGE_STEPS - 1
    nof = _EDGE_STEPS
    return pl.pallas_call(
        _fused_body,
        grid=(_EDGE_STEPS + _NODE_STEPS,),
        in_specs=[
            pl.BlockSpec((_ED, _EDGE_TILE_T),
                         lambda i: (0, jnp.minimum(i, ecl))),
            pl.BlockSpec((_ED, _ED), lambda i: (0, 0)),
            pl.BlockSpec((_ED, 1), lambda i: (0, 0)),
            pl.BlockSpec((_NODE_TILE, _D),
                         lambda i: (jnp.maximum(i - nof, 0), 0)),
            pl.BlockSpec((_NC, _NODE_TILE, 1),
                         lambda i: (0, jnp.maximum(i - nof, 0), 0)),
            pl.BlockSpec((4, _D, _D), lambda i: (0, 0, 0)),
            pl.BlockSpec((4, _D, _D), lambda i: (0, 0, 0)),
            pl.BlockSpec((_D, 2 * _D), lambda i: (0, 0)),
            pl.BlockSpec((1, _D), lambda i: (0, 0)),
            pl.BlockSpec((_D, _D), lambda i: (0, 0)),
            pl.BlockSpec((1, _D), lambda i: (0, 0)),
        ],
        out_specs=[
            pl.BlockSpec((_ED, _EDGE_TILE_T),
                         lambda i: (0, jnp.minimum(i, ecl))),
            pl.BlockSpec((_NODE_TILE, _D),
                         lambda i: (jnp.maximum(i - nof, 0), 0)),
        ],
        out_shape=[
            jax.ShapeDtypeStruct((_ED, _E), jnp.float32),
            jax.ShapeDtypeStruct((_N, _D), jnp.float32),
        ],
        scratch_shapes=[pltpu.VMEM((_D, _D), jnp.float32)],
    )(efT, we, be, x, degs, watt, wrel, wout, bout, wfc, bfc)


def kernel(node_features, edge_features, edge_index, node_att_W, rel_att_W,
           rel_att_W1, rel_att_b1, rel_att_W2, rel_att_b2, node_out_W,
           node_out_b, node_fc_W, node_fc_b, edge_fc_W, edge_fc_b):
    dst_pad = jnp.concatenate(
        [edge_index[1],
         jnp.full((_ROWS_PAD * _CHUNK - _E,), _DEAD, jnp.int32)]
    ).reshape(_ROWS_PAD, _CHUNK)
    degs = _deg_counts_fn()(dst_pad).reshape(_NC, _ACC, 1)
    watt = node_att_W.reshape(node_att_W.shape[1], _D, _D)
    wrel = rel_att_W.reshape(rel_att_W.shape[1], _D, _D)
    new_edgesT, new_nodes = _fused_call(
        edge_features.T, edge_fc_W, edge_fc_b.reshape(_ED, 1),
        node_features, degs, watt, wrel, node_out_W,
        node_out_b.reshape(1, _D), node_fc_W, node_fc_b.reshape(1, _D))
    return new_nodes, new_edgesT.T


# speedup vs baseline: 1.0968x; 1.0968x over previous
import functools

import jax
import jax.numpy as jnp
from jax import lax
from jax.experimental import pallas as pl
from jax.experimental.pallas import tpu as pltpu
from jax.experimental.pallas import tpu_sc as plsc

_N = 10000
_E = 160000
_D = 128
_ED = 16

_NC = 2
_NS = 16
_NW = _NC * _NS
_ACC = 10240
_CHUNK = 128
_ROWS = _E // _CHUNK
_ROWS_PW = 40
_FIRE = 20
_ROWS_PAD = _NW * _ROWS_PW
_DEAD = _N + 16
_ZLEN = _ACC // _NS


def _deg_body(dst_hbm, out_hbm, idx_v, ones_v, zeros_v, acc_shared, sem):
    c = lax.axis_index("c")
    s = lax.axis_index("s")
    wid = c * _NS + s
    for i in range(_CHUNK // 16):
        ones_v[pl.ds(i * 16, 16)] = jnp.ones((16,), jnp.float32)
    for i in range(_ZLEN // 16):
        zeros_v[pl.ds(i * 16, 16)] = jnp.zeros((16,), jnp.float32)
    pltpu.sync_copy(dst_hbm.at[pl.ds(wid * _ROWS_PW, _ROWS_PW)], idx_v)
    pltpu.sync_copy(zeros_v, acc_shared.at[pl.ds(s * _ZLEN, _ZLEN)])
    plsc.subcore_barrier()

    def _block(b, carry):
        copies = [
            pltpu.async_copy(
                ones_v, acc_shared.at[idx_v.at[b * _FIRE + j]], sem, add=True)
            for j in range(_FIRE)
        ]
        for cp in copies:
            cp.wait()
        return carry

    lax.fori_loop(0, _ROWS_PW // _FIRE, _block, 0)
    plsc.subcore_barrier()
    pltpu.sync_copy(acc_shared.at[pl.ds(s * _ZLEN, _ZLEN)],
                    out_hbm.at[pl.ds(c * _ACC + s * _ZLEN, _ZLEN)])


@functools.cache
def _deg_counts_fn():
    return pl.kernel(
        _deg_body,
        out_type=jax.ShapeDtypeStruct((_NC * _ACC,), jnp.float32),
        mesh=plsc.VectorSubcoreMesh(core_axis_name="c", subcore_axis_name="s"),
        scratch_types=[
            pltpu.VMEM((_ROWS_PW, _CHUNK), jnp.int32),
            pltpu.VMEM((_CHUNK,), jnp.float32),
            pltpu.VMEM((_ZLEN,), jnp.float32),
            pltpu.VMEM_SHARED((_ACC,), jnp.float32),
            pltpu.SemaphoreType.DMA,
        ],
    )


_NODE_TILE = 2000
_CONTRACT_T = (((1,), (1,)), ((), ()))


def _node_body(x_ref, deg_ref, watt_ref, wrel_ref, wout_ref, bout_ref,
               wfc_ref, bfc_ref, o_ref, wcomb_ref):
    @pl.when(pl.program_id(0) == 0)
    def _fold():
        kk = watt_ref.shape[0]
        mm = wrel_ref.shape[0]
        wk = watt_ref[0]
        for k in range(1, kk):
            wk = wk + watt_ref[k]
        wm = wrel_ref[0]
        for m in range(1, mm):
            wm = wm + wrel_ref[m]
        c1 = wout_ref[:, :_D]
        c2 = wout_ref[:, _D:]
        wcomb_ref[...] = (
            lax.dot_general(wk, c1, _CONTRACT_T,
                            preferred_element_type=jnp.float32) / kk
            + lax.dot_general(wm, c2, _CONTRACT_T,
                              preferred_element_type=jnp.float32) / mm)

    h = x_ref[...]
    t = jnp.maximum(
        jnp.dot(h, wcomb_ref[...], preferred_element_type=jnp.float32)
        + bout_ref[...], 0.0)
    d = deg_ref[0] + deg_ref[1]
    h2 = jnp.where(d > 0.0, t, h)
    o = jnp.maximum(
        lax.dot_general(h2, wfc_ref[...], _CONTRACT_T,
                        preferred_element_type=jnp.float32)
        + bfc_ref[...], 0.0) + h
    o_ref[...] = o


def _node_call(x, degs, watt, wrel, wout, bout, wfc, bfc):
    grid = (_N // _NODE_TILE,)
    return pl.pallas_call(
        _node_body,
        grid=grid,
        in_specs=[
            pl.BlockSpec((_NODE_TILE, _D), lambda i: (i, 0)),
            pl.BlockSpec((_NC, _NODE_TILE, 1), lambda i: (0, i, 0)),
            pl.BlockSpec(watt.shape, lambda i: (0, 0, 0)),
            pl.BlockSpec(wrel.shape, lambda i: (0, 0, 0)),
            pl.BlockSpec((_D, 2 * _D), lambda i: (0, 0)),
            pl.BlockSpec((1, _D), lambda i: (0, 0)),
            pl.BlockSpec((_D, _D), lambda i: (0, 0)),
            pl.BlockSpec((1, _D), lambda i: (0, 0)),
        ],
        out_specs=pl.BlockSpec((_NODE_TILE, _D), lambda i: (i, 0)),
        out_shape=jax.ShapeDtypeStruct((_N, _D), jnp.float32),
        scratch_shapes=[pltpu.VMEM((_D, _D), jnp.float32)],
    )(x, degs, watt, wrel, wout, bout, wfc, bfc)


_EDGE_TILE_T = 32000


def _edge_body_t(x_ref, w_ref, b_ref, o_ref):
    x = x_ref[...]
    o_ref[...] = jnp.maximum(
        lax.dot_general(w_ref[...], x, (((1,), (0,)), ((), ())),
                        preferred_element_type=jnp.float32)
        + b_ref[...], 0.0) + x


def _edge_call_t(efT, w, b):
    grid = (_E // _EDGE_TILE_T,)
    return pl.pallas_call(
        _edge_body_t,
        grid=grid,
        in_specs=[
            pl.BlockSpec((_ED, _EDGE_TILE_T), lambda i: (0, i)),
            pl.BlockSpec((_ED, _ED), lambda i: (0, 0)),
            pl.BlockSpec((_ED, 1), lambda i: (0, 0)),
        ],
        out_specs=pl.BlockSpec((_ED, _EDGE_TILE_T), lambda i: (0, i)),
        out_shape=jax.ShapeDtypeStruct((_ED, _E), jnp.float32),
    )(efT, w, b)


def kernel(node_features, edge_features, edge_index, node_att_W, rel_att_W,
           rel_att_W1, rel_att_b1, rel_att_W2, rel_att_b2, node_out_W,
           node_out_b, node_fc_W, node_fc_b, edge_fc_W, edge_fc_b):
    dst_pad = jnp.concatenate(
        [edge_index[1],
         jnp.full((_ROWS_PAD * _CHUNK - _E,), _DEAD, jnp.int32)]
    ).reshape(_ROWS_PAD, _CHUNK)
    degs = _deg_counts_fn()(dst_pad).reshape(_NC, _ACC, 1)
    new_edges = _edge_call_t(edge_features.T, edge_fc_W,
                             edge_fc_b.reshape(_ED, 1)).T
    watt = node_att_W.reshape(node_att_W.shape[1], _D, _D)
    wrel = rel_att_W.reshape(rel_att_W.shape[1], _D, _D)
    new_nodes = _node_call(node_features, degs, watt, wrel, node_out_W,
                           node_out_b.reshape(1, _D), node_fc_W,
                           node_fc_b.reshape(1, _D))
    return new_nodes, new_edges


# speedup vs baseline: 1.1000x; 1.0029x over previous
import functools

import jax
import jax.numpy as jnp
from jax import lax
from jax.experimental import pallas as pl
from jax.experimental.pallas import tpu as pltpu
from jax.experimental.pallas import tpu_sc as plsc

_N = 10000
_E = 160000
_D = 128
_ED = 16

_NC = 2
_NS = 16
_NW = _NC * _NS
_ACC = 10240
_CHUNK = 128
_ROWS = _E // _CHUNK
_ROWS_PW = 40
_FIRE = 20
_ROWS_PAD = _NW * _ROWS_PW
_DEAD = _N + 16
_ZLEN = _ACC // _NS


def _deg_body(dst_hbm, out_hbm, idx_v, ones_v, zeros_v, acc_shared, sem):
    c = lax.axis_index("c")
    s = lax.axis_index("s")
    wid = c * _NS + s
    for i in range(_CHUNK // 16):
        ones_v[pl.ds(i * 16, 16)] = jnp.ones((16,), jnp.float32)
    for i in range(_ZLEN // 16):
        zeros_v[pl.ds(i * 16, 16)] = jnp.zeros((16,), jnp.float32)
    pltpu.sync_copy(dst_hbm.at[pl.ds(wid * _ROWS_PW, _ROWS_PW)], idx_v)
    pltpu.sync_copy(zeros_v, acc_shared.at[pl.ds(s * _ZLEN, _ZLEN)])
    plsc.subcore_barrier()

    def _block(b, carry):
        copies = [
            pltpu.async_copy(
                ones_v, acc_shared.at[idx_v.at[b * _FIRE + j]], sem, add=True)
            for j in range(_FIRE)
        ]
        for cp in copies:
            cp.wait()
        return carry

    lax.fori_loop(0, _ROWS_PW // _FIRE, _block, 0)
    plsc.subcore_barrier()
    pltpu.sync_copy(acc_shared.at[pl.ds(s * _ZLEN, _ZLEN)],
                    out_hbm.at[pl.ds(c * _ACC + s * _ZLEN, _ZLEN)])


@functools.cache
def _deg_counts_fn():
    return pl.kernel(
        _deg_body,
        out_type=jax.ShapeDtypeStruct((_NC * _ACC,), jnp.float32),
        mesh=plsc.VectorSubcoreMesh(core_axis_name="c", subcore_axis_name="s"),
        scratch_types=[
            pltpu.VMEM((_ROWS_PW, _CHUNK), jnp.int32),
            pltpu.VMEM((_CHUNK,), jnp.float32),
            pltpu.VMEM((_ZLEN,), jnp.float32),
            pltpu.VMEM_SHARED((_ACC,), jnp.float32),
            pltpu.SemaphoreType.DMA,
        ],
    )


_NODE_TILE = 2000
_CONTRACT_T = (((1,), (1,)), ((), ()))


def _node_body(x_ref, deg_ref, watt_ref, wrel_ref, wout_ref, bout_ref,
               wfc_ref, bfc_ref, o_ref, wcomb_ref):
    @pl.when(pl.program_id(0) == 0)
    def _fold():
        kk = watt_ref.shape[0]
        mm = wrel_ref.shape[0]
        wk = watt_ref[0]
        for k in range(1, kk):
            wk = wk + watt_ref[k]
        wm = wrel_ref[0]
        for m in range(1, mm):
            wm = wm + wrel_ref[m]
        c1 = wout_ref[:, :_D]
        c2 = wout_ref[:, _D:]
        wcomb_ref[...] = (
            lax.dot_general(wk, c1, _CONTRACT_T,
                            preferred_element_type=jnp.float32) / kk
            + lax.dot_general(wm, c2, _CONTRACT_T,
                              preferred_element_type=jnp.float32) / mm)

    h = x_ref[...]
    t = jnp.maximum(
        jnp.dot(h, wcomb_ref[...], preferred_element_type=jnp.float32)
        + bout_ref[...], 0.0)
    d = deg_ref[0] + deg_ref[1]
    h2 = jnp.where(d > 0.0, t, h)
    o = jnp.maximum(
        lax.dot_general(h2, wfc_ref[...], _CONTRACT_T,
                        preferred_element_type=jnp.float32)
        + bfc_ref[...], 0.0) + h
    o_ref[...] = o


def _node_call(x, degs, watt, wrel, wout, bout, wfc, bfc):
    grid = (_N // _NODE_TILE,)
    return pl.pallas_call(
        _node_body,
        grid=grid,
        in_specs=[
            pl.BlockSpec((_NODE_TILE, _D), lambda i: (i, 0)),
            pl.BlockSpec((_NC, _NODE_TILE, 1), lambda i: (0, i, 0)),
            pl.BlockSpec(watt.shape, lambda i: (0, 0, 0)),
            pl.BlockSpec(wrel.shape, lambda i: (0, 0, 0)),
            pl.BlockSpec((_D, 2 * _D), lambda i: (0, 0)),
            pl.BlockSpec((1, _D), lambda i: (0, 0)),
            pl.BlockSpec((_D, _D), lambda i: (0, 0)),
            pl.BlockSpec((1, _D), lambda i: (0, 0)),
        ],
        out_specs=pl.BlockSpec((_NODE_TILE, _D), lambda i: (i, 0)),
        out_shape=jax.ShapeDtypeStruct((_N, _D), jnp.float32),
        scratch_shapes=[pltpu.VMEM((_D, _D), jnp.float32)],
    )(x, degs, watt, wrel, wout, bout, wfc, bfc)


_EDGE_TILE_T = 32000


def _edge_body_t(x_ref, w_ref, b_ref, o_ref):
    x = x_ref[...]
    o_ref[...] = jnp.maximum(
        lax.dot_general(w_ref[...], x, (((1,), (0,)), ((), ())),
                        preferred_element_type=jnp.float32)
        + b_ref[...], 0.0) + x


def _edge_call_t(efT, w, b):
    grid = (_E // _EDGE_TILE_T,)
    return pl.pallas_call(
        _edge_body_t,
        grid=grid,
        in_specs=[
            pl.BlockSpec((_ED, _EDGE_TILE_T), lambda i: (0, i)),
            pl.BlockSpec((_ED, _ED), lambda i: (0, 0)),
            pl.BlockSpec((_ED, 1), lambda i: (0, 0)),
        ],
        out_specs=pl.BlockSpec((_ED, _EDGE_TILE_T), lambda i: (0, i)),
        out_shape=jax.ShapeDtypeStruct((_ED, _E), jnp.float32),
    )(efT, w, b)


def kernel(node_features, edge_features, edge_index, node_att_W, rel_att_W,
           rel_att_W1, rel_att_b1, rel_att_W2, rel_att_b2, node_out_W,
           node_out_b, node_fc_W, node_fc_b, edge_fc_W, edge_fc_b):
    dst_pad = jnp.concatenate(
        [edge_index[1],
         jnp.full((_ROWS_PAD * _CHUNK - _E,), _DEAD, jnp.int32)]
    ).reshape(_ROWS_PAD, _CHUNK)
    new_edges = _edge_call_t(edge_features.T, edge_fc_W,
                             edge_fc_b.reshape(_ED, 1)).T
    degs = _deg_counts_fn()(dst_pad).reshape(_NC, _ACC, 1)
    watt = node_att_W.reshape(node_att_W.shape[1], _D, _D)
    wrel = rel_att_W.reshape(rel_att_W.shape[1], _D, _D)
    new_nodes = _node_call(node_features, degs, watt, wrel, node_out_W,
                           node_out_b.reshape(1, _D), node_fc_W,
                           node_fc_b.reshape(1, _D))
    return new_nodes, new_edges
